# Initial kernel scaffold; baseline (speedup 1.0000x reference)
import jax
import jax.numpy as jnp
from jax.experimental import pallas as pl


def kernel(node_attrs, node_feats, edge_attrs, edge_feats, edge_index, W_up, W1, W2, W3, W4, W_lin, W_skip):
    raise NotImplementedError("write your pallas kernel here")


# speedup vs baseline: 2.1574x; 2.1574x over previous
import functools
import math

import jax
import jax.numpy as jnp
from jax import lax
from jax.experimental import pallas as pl
from jax.experimental.pallas import tpu as pltpu
from jax.experimental.pallas import tpu_sc as plsc

N = 10000
E = 320000
D = 128
A = 10
R = 8
H = 64
AVG_NUM_NEIGHBORS = 32.0

NUM_SC = 2
NUM_TILES = 16
NW = NUM_SC * NUM_TILES
E_PER_TILE = E // NW
CHUNK = 80
N_CHUNKS = E_PER_TILE // CHUNK
N_PAD = 10240
ROWS_PER_TILE = N_PAD // NUM_TILES


def _mlp_body(ef_ref, ea_ref, w1_ref, w2_ref, w3_ref, w4_ref, out_ref):
    h = jnp.dot(ef_ref[...], w1_ref[...], preferred_element_type=jnp.float32)
    h = h * (1.0 / math.sqrt(R))
    h = h * jax.nn.sigmoid(h)
    h = jnp.dot(h, w2_ref[...], preferred_element_type=jnp.float32)
    h = h * (1.0 / math.sqrt(H))
    h = h * jax.nn.sigmoid(h)
    h = jnp.dot(h, w3_ref[...], preferred_element_type=jnp.float32)
    h = h * (1.0 / math.sqrt(H))
    h = h * jax.nn.sigmoid(h)
    tp = jnp.dot(h, w4_ref[...], preferred_element_type=jnp.float32)
    tp = tp * (1.0 / math.sqrt(H))
    out_ref[...] = tp * ea_ref[...]


def _edge_mlp(edge_feats, edge_attrs, W1, W2, W3, W4):
    BE = 4000
    grid = E // BE
    return pl.pallas_call(
        _mlp_body,
        grid=(grid,),
        in_specs=[
            pl.BlockSpec((BE, R), lambda i: (i, 0)),
            pl.BlockSpec((BE, 1), lambda i: (i, 0)),
            pl.BlockSpec((R, H), lambda i: (0, 0)),
            pl.BlockSpec((H, H), lambda i: (0, 0)),
            pl.BlockSpec((H, H), lambda i: (0, 0)),
            pl.BlockSpec((H, D), lambda i: (0, 0)),
        ],
        out_specs=pl.BlockSpec((BE, D), lambda i: (i, 0)),
        out_shape=jax.ShapeDtypeStruct((E, D), jnp.float32),
    )(edge_feats, edge_attrs, W1, W2, W3, W4)


def _up_body(nf_ref, w_ref, out_ref):
    out_ref[...] = jnp.dot(
        nf_ref[...], w_ref[...], preferred_element_type=jnp.float32
    ) * (1.0 / math.sqrt(D))


def _linear_up(node_feats, W_up):
    return pl.pallas_call(
        _up_body,
        out_shape=jax.ShapeDtypeStruct((N, D), jnp.float32),
    )(node_feats, W_up)


def _sc_body(x_hbm, tp_hbm, snd_hbm, rcv_hbm, zero_hbm, out_hbm,
             idx_s, idx_r, xrows, tprows, acc, sem):
    c = lax.axis_index("c")
    s = lax.axis_index("s")
    wid = c * NUM_TILES + s

    pltpu.sync_copy(zero_hbm, acc.at[pl.ds(s * ROWS_PER_TILE, ROWS_PER_TILE)])
    plsc.subcore_barrier()

    base0 = wid * E_PER_TILE

    def chunk_body(t, carry):
        base = base0 + t * CHUNK
        pltpu.sync_copy(snd_hbm.at[pl.ds(base, CHUNK)], idx_s)
        pltpu.sync_copy(rcv_hbm.at[pl.ds(base, CHUNK)], idx_r)
        pltpu.sync_copy(tp_hbm.at[pl.ds(base, CHUNK)], tprows)
        pltpu.async_copy(x_hbm.at[idx_s], xrows, sem).wait()

        def mul_body(i, c2):
            for j in range(D // 16):
                sl = pl.ds(j * 16, 16)
                xrows[i, sl] = xrows[i, sl] * tprows[i, sl]
            return c2

        lax.fori_loop(0, CHUNK, mul_body, 0)
        pltpu.sync_copy(xrows, acc.at[idx_r], add=True)
        return carry

    lax.fori_loop(0, N_CHUNKS, chunk_body, 0)
    plsc.subcore_barrier()

    pltpu.sync_copy(acc.at[pl.ds(s * ROWS_PER_TILE, ROWS_PER_TILE)],
                    out_hbm.at[c, pl.ds(s * ROWS_PER_TILE, ROWS_PER_TILE)])


def _sc_scatter(x, tp_scaled, sender, receiver, zeros):
    mesh = plsc.VectorSubcoreMesh(core_axis_name="c", subcore_axis_name="s")
    f = functools.partial(
        pl.kernel,
        out_type=jax.ShapeDtypeStruct((NUM_SC, N_PAD, D), jnp.float32),
        mesh=mesh,
        scratch_types=[
            pltpu.VMEM((CHUNK,), jnp.int32),
            pltpu.VMEM((CHUNK,), jnp.int32),
            pltpu.VMEM((CHUNK, D), jnp.float32),
            pltpu.VMEM((CHUNK, D), jnp.float32),
            pltpu.VMEM_SHARED((N_PAD, D), jnp.float32),
            pltpu.SemaphoreType.DMA,
        ],
    )(_sc_body)
    return f(x, tp_scaled, sender, receiver, zeros)


def _final_body(parts_ref, na_ref, wlin_ref, wskip_ref, out_ref):
    m = parts_ref[0] + parts_ref[1]
    z = jnp.dot(m, wlin_ref[...], preferred_element_type=jnp.float32)
    z = z * (1.0 / (math.sqrt(D) * AVG_NUM_NEIGHBORS))
    acc = jnp.zeros(out_ref.shape, jnp.float32)
    for v in range(A):
        acc = acc + jnp.dot(
            z, wskip_ref[:, v, :], preferred_element_type=jnp.float32
        ) * na_ref[:, v:v + 1]
    out_ref[...] = acc * (1.0 / math.sqrt(D * A))


def _final(parts, node_attrs, W_lin, W_skip):
    BN = 2000
    grid = N // BN
    return pl.pallas_call(
        _final_body,
        grid=(grid,),
        in_specs=[
            pl.BlockSpec((NUM_SC, BN, D), lambda i: (0, i, 0)),
            pl.BlockSpec((BN, A), lambda i: (i, 0)),
            pl.BlockSpec((D, D), lambda i: (0, 0)),
            pl.BlockSpec((D, A, D), lambda i: (0, 0, 0)),
        ],
        out_specs=pl.BlockSpec((BN, D), lambda i: (i, 0)),
        out_shape=jax.ShapeDtypeStruct((N, D), jnp.float32),
    )(parts, node_attrs, W_lin, W_skip)


def kernel(node_attrs, node_feats, edge_attrs, edge_feats, edge_index,
           W_up, W1, W2, W3, W4, W_lin, W_skip):
    edge_index = edge_index.astype(jnp.int32)
    sender = edge_index[0]
    receiver = edge_index[1]
    tp_scaled = _edge_mlp(edge_feats, edge_attrs, W1, W2, W3, W4)
    x = _linear_up(node_feats, W_up)
    zeros = jnp.zeros((ROWS_PER_TILE, D), jnp.float32)
    parts = _sc_scatter(x, tp_scaled, sender, receiver, zeros)
    return _final(parts[:, :N], node_attrs, W_lin, W_skip)


# speedup vs baseline: 2.9284x; 1.3574x over previous
import functools
import math

import jax
import jax.numpy as jnp
from jax import lax
from jax.experimental import pallas as pl
from jax.experimental.pallas import tpu as pltpu
from jax.experimental.pallas import tpu_sc as plsc

N = 10000
E = 320000
D = 128
A = 10
R = 8
H = 64
AVG_NUM_NEIGHBORS = 32.0

NUM_SC = 2
NUM_TILES = 16
NW = NUM_SC * NUM_TILES
CHUNK = 80
E_PER_TILE = E // NW
N_CHUNKS = E_PER_TILE // CHUNK
NQ = 31
NIDX = 4
N_PAD = 10240
ROWS_PER_TILE = N_PAD // NUM_TILES


def _mlp_body(ef_ref, ea_ref, w1_ref, w2_ref, w3_ref, w4_ref, out_ref):
    h = jnp.dot(ef_ref[...], w1_ref[...], preferred_element_type=jnp.float32)
    h = h * (1.0 / math.sqrt(R))
    h = h * jax.nn.sigmoid(h)
    h = jnp.dot(h, w2_ref[...], preferred_element_type=jnp.float32)
    h = h * (1.0 / math.sqrt(H))
    h = h * jax.nn.sigmoid(h)
    h = jnp.dot(h, w3_ref[...], preferred_element_type=jnp.float32)
    h = h * (1.0 / math.sqrt(H))
    h = h * jax.nn.sigmoid(h)
    tp = jnp.dot(h, w4_ref[...], preferred_element_type=jnp.float32)
    tp = tp * (1.0 / math.sqrt(H))
    out_ref[...] = tp * ea_ref[...]


def _edge_mlp(edge_feats, edge_attrs, W1, W2, W3, W4):
    BE = 4000
    grid = E // BE
    return pl.pallas_call(
        _mlp_body,
        grid=(grid,),
        in_specs=[
            pl.BlockSpec((BE, R), lambda i: (i, 0)),
            pl.BlockSpec((BE, 1), lambda i: (i, 0)),
            pl.BlockSpec((R, H), lambda i: (0, 0)),
            pl.BlockSpec((H, H), lambda i: (0, 0)),
            pl.BlockSpec((H, H), lambda i: (0, 0)),
            pl.BlockSpec((H, D), lambda i: (0, 0)),
        ],
        out_specs=pl.BlockSpec((BE, D), lambda i: (i, 0)),
        out_shape=jax.ShapeDtypeStruct((E, D), jnp.float32),
    )(edge_feats, edge_attrs, W1, W2, W3, W4)


def _up_body(nf_ref, w_ref, out_ref):
    out_ref[...] = jnp.dot(
        nf_ref[...], w_ref[...], preferred_element_type=jnp.float32
    ) * (1.0 / math.sqrt(D))


def _linear_up(node_feats, W_up):
    return pl.pallas_call(
        _up_body,
        out_shape=jax.ShapeDtypeStruct((N, D), jnp.float32),
    )(node_feats, W_up)


def _sc_body(x_hbm, tp_hbm, snd_hbm, rcv_hbm, zero_hbm, out_hbm,
             idxb, xrows, tprows, acc,
             si, st0, st1, sg0, sg1, ss0, ss1):
    sem_tp = (st0, st1)
    sem_g = (sg0, sg1)
    sem_s = (ss0, ss1)
    c = lax.axis_index("c")
    s = lax.axis_index("s")
    wid = c * NUM_TILES + s

    base0 = wid * E_PER_TILE

    def start_idx_tp(t, ib, xb):
        base = base0 + t * CHUNK
        pltpu.async_copy(snd_hbm.at[pl.ds(base, CHUNK)], idxb.at[ib, 0], si)
        pltpu.async_copy(rcv_hbm.at[pl.ds(base, CHUNK)], idxb.at[ib, 1], si)
        pltpu.async_copy(tp_hbm.at[pl.ds(base, CHUNK)], tprows.at[xb],
                         sem_tp[xb])

    def wait_idx(ib):
        pltpu.make_async_copy(snd_hbm.at[pl.ds(base0, CHUNK)],
                              idxb.at[ib, 0], si).wait()
        pltpu.make_async_copy(rcv_hbm.at[pl.ds(base0, CHUNK)],
                              idxb.at[ib, 1], si).wait()

    def wait_tp(xb):
        pltpu.make_async_copy(tp_hbm.at[pl.ds(base0, CHUNK)],
                              tprows.at[xb], sem_tp[xb]).wait()

    def start_gather(ib, xb):
        pltpu.async_copy(x_hbm.at[idxb.at[ib, 0]], xrows.at[xb], sem_g[xb])

    def wait_gather(ib, xb):
        pltpu.make_async_copy(x_hbm.at[idxb.at[ib, 0]], xrows.at[xb],
                              sem_g[xb]).wait()

    def start_scatter(ib, xb):
        pltpu.async_copy(xrows.at[xb], acc.at[idxb.at[ib, 1]], sem_s[xb],
                         add=True)

    def wait_scatter(ib, xb):
        pltpu.make_async_copy(xrows.at[xb], acc.at[idxb.at[ib, 1]],
                              sem_s[xb]).wait()

    def multiply(xb):
        def mul_body(i, carry):
            for j in range(D // 16):
                sl = pl.ds(j * 16, 16)
                xrows[xb, i, sl] = xrows[xb, i, sl] * tprows[xb, i, sl]
            return carry
        lax.fori_loop(0, CHUNK, mul_body, 0)

    def chunk_step(t, ib, xb, prefetch=True, first_guard=None):
        in1 = (ib + 1) % NIDX
        xn = xb ^ 1
        if prefetch:
            start_idx_tp(t + 1, in1, xn)
        wait_tp(xb)
        wait_gather(ib, xb)
        multiply(xb)
        if first_guard is None:
            wait_scatter((ib - 1) % NIDX, xn)
        else:
            @pl.when(first_guard)
            def _():
                wait_scatter((ib - 1) % NIDX, xn)
        if prefetch:
            wait_idx(in1)
            start_gather(in1, xn)
        start_scatter(ib, xb)

    start_idx_tp(0, 0, 0)
    pltpu.sync_copy(zero_hbm, acc.at[pl.ds(s * ROWS_PER_TILE, ROWS_PER_TILE)])
    plsc.subcore_barrier()
    wait_idx(0)
    start_gather(0, 0)

    def quad_body(q, carry):
        chunk_step(4 * q + 0, 0, 0, first_guard=q >= 1)
        chunk_step(4 * q + 1, 1, 1)
        chunk_step(4 * q + 2, 2, 0)
        chunk_step(4 * q + 3, 3, 1)
        return carry

    lax.fori_loop(0, NQ, quad_body, 0)

    wait_tp(0)
    wait_gather(0, 0)
    multiply(0)
    wait_scatter(3, 1)
    start_scatter(0, 0)
    wait_scatter(0, 0)
    plsc.subcore_barrier()

    pltpu.sync_copy(acc.at[pl.ds(s * ROWS_PER_TILE, ROWS_PER_TILE)],
                    out_hbm.at[c, pl.ds(s * ROWS_PER_TILE, ROWS_PER_TILE)])


def _sc_scatter(x, tp_scaled, sender, receiver, zeros):
    mesh = plsc.VectorSubcoreMesh(core_axis_name="c", subcore_axis_name="s")
    f = functools.partial(
        pl.kernel,
        out_type=jax.ShapeDtypeStruct((NUM_SC, N_PAD, D), jnp.float32),
        mesh=mesh,
        scratch_types=[
            pltpu.VMEM((NIDX, 2, CHUNK), jnp.int32),
            pltpu.VMEM((2, CHUNK, D), jnp.float32),
            pltpu.VMEM((2, CHUNK, D), jnp.float32),
            pltpu.VMEM_SHARED((N_PAD, D), jnp.float32),
            pltpu.SemaphoreType.DMA,
            pltpu.SemaphoreType.DMA,
            pltpu.SemaphoreType.DMA,
            pltpu.SemaphoreType.DMA,
            pltpu.SemaphoreType.DMA,
            pltpu.SemaphoreType.DMA,
            pltpu.SemaphoreType.DMA,
        ],
    )(_sc_body)
    return f(x, tp_scaled, sender, receiver, zeros)


def _final_body(parts_ref, na_ref, wlin_ref, wskip_ref, out_ref):
    m = parts_ref[0] + parts_ref[1]
    z = jnp.dot(m, wlin_ref[...], preferred_element_type=jnp.float32)
    z = z * (1.0 / (math.sqrt(D) * AVG_NUM_NEIGHBORS))
    acc = jnp.zeros(out_ref.shape, jnp.float32)
    for v in range(A):
        acc = acc + jnp.dot(
            z, wskip_ref[:, v, :], preferred_element_type=jnp.float32
        ) * na_ref[:, v:v + 1]
    out_ref[...] = acc * (1.0 / math.sqrt(D * A))


def _final(parts, node_attrs, W_lin, W_skip):
    BN = 2000
    grid = N // BN
    return pl.pallas_call(
        _final_body,
        grid=(grid,),
        in_specs=[
            pl.BlockSpec((NUM_SC, BN, D), lambda i: (0, i, 0)),
            pl.BlockSpec((BN, A), lambda i: (i, 0)),
            pl.BlockSpec((D, D), lambda i: (0, 0)),
            pl.BlockSpec((D, A, D), lambda i: (0, 0, 0)),
        ],
        out_specs=pl.BlockSpec((BN, D), lambda i: (i, 0)),
        out_shape=jax.ShapeDtypeStruct((N, D), jnp.float32),
    )(parts, node_attrs, W_lin, W_skip)


def kernel(node_attrs, node_feats, edge_attrs, edge_feats, edge_index,
           W_up, W1, W2, W3, W4, W_lin, W_skip):
    edge_index = edge_index.astype(jnp.int32)
    tp_scaled = _edge_mlp(edge_feats, edge_attrs, W1, W2, W3, W4)
    x = _linear_up(node_feats, W_up)
    zeros = jnp.zeros((ROWS_PER_TILE, D), jnp.float32)
    parts = _sc_scatter(x, tp_scaled, edge_index[0], edge_index[1], zeros)
    return _final(parts[:, :N], node_attrs, W_lin, W_skip)


# speedup vs baseline: 2.9453x; 1.0058x over previous
import functools
import math

import jax
import jax.numpy as jnp
from jax import lax
from jax.experimental import pallas as pl
from jax.experimental.pallas import tpu as pltpu
from jax.experimental.pallas import tpu_sc as plsc

N = 10000
E = 320000
D = 128
A = 10
R = 8
H = 64
AVG_NUM_NEIGHBORS = 32.0

NUM_SC = 2
NUM_TILES = 16
NW = NUM_SC * NUM_TILES
CHUNK = 80
E_PER_TILE = E // NW
N_CHUNKS = E_PER_TILE // CHUNK
NQ = 31
NIDX = 4
N_PAD = 10240
ROWS_PER_TILE = N_PAD // NUM_TILES


def _mlp_body(ef_ref, ea_ref, w1_ref, w2_ref, w3_ref, w4_ref, out_ref):
    h = jnp.dot(ef_ref[...], w1_ref[...], preferred_element_type=jnp.float32)
    h = h * (1.0 / math.sqrt(R))
    h = h * jax.nn.sigmoid(h)
    h = jnp.dot(h.astype(jnp.bfloat16), w2_ref[...],
                preferred_element_type=jnp.float32)
    h = h * (1.0 / math.sqrt(H))
    h = h * jax.nn.sigmoid(h)
    h = jnp.dot(h.astype(jnp.bfloat16), w3_ref[...],
                preferred_element_type=jnp.float32)
    h = h * (1.0 / math.sqrt(H))
    h = h * jax.nn.sigmoid(h)
    tp = jnp.dot(h.astype(jnp.bfloat16), w4_ref[...],
                 preferred_element_type=jnp.float32)
    tp = tp * (1.0 / math.sqrt(H))
    out_ref[...] = tp * ea_ref[...]


def _edge_mlp(edge_feats, edge_attrs, W1, W2, W3, W4):
    W2 = W2.astype(jnp.bfloat16)
    W3 = W3.astype(jnp.bfloat16)
    W4 = W4.astype(jnp.bfloat16)
    BE = 4000
    grid = E // BE
    return pl.pallas_call(
        _mlp_body,
        grid=(grid,),
        in_specs=[
            pl.BlockSpec((BE, R), lambda i: (i, 0)),
            pl.BlockSpec((BE, 1), lambda i: (i, 0)),
            pl.BlockSpec((R, H), lambda i: (0, 0)),
            pl.BlockSpec((H, H), lambda i: (0, 0)),
            pl.BlockSpec((H, H), lambda i: (0, 0)),
            pl.BlockSpec((H, D), lambda i: (0, 0)),
        ],
        out_specs=pl.BlockSpec((BE, D), lambda i: (i, 0)),
        out_shape=jax.ShapeDtypeStruct((E, D), jnp.float32),
    )(edge_feats, edge_attrs, W1, W2, W3, W4)


def _up_body(nf_ref, w_ref, out_ref):
    out_ref[...] = jnp.dot(
        nf_ref[...], w_ref[...], preferred_element_type=jnp.float32
    ) * (1.0 / math.sqrt(D))


def _linear_up(node_feats, W_up):
    return pl.pallas_call(
        _up_body,
        out_shape=jax.ShapeDtypeStruct((N, D), jnp.float32),
    )(node_feats, W_up)


def _sc_body(x_hbm, tp_hbm, snd_hbm, rcv_hbm, zero_hbm, out_hbm,
             idxb, xrows, tprows, acc,
             si, st0, st1, sg0, sg1, ss0, ss1):
    sem_tp = (st0, st1)
    sem_g = (sg0, sg1)
    sem_s = (ss0, ss1)
    c = lax.axis_index("c")
    s = lax.axis_index("s")
    wid = c * NUM_TILES + s

    base0 = wid * E_PER_TILE

    def start_idx_tp(t, ib, xb):
        base = base0 + t * CHUNK
        pltpu.async_copy(snd_hbm.at[pl.ds(base, CHUNK)], idxb.at[ib, 0], si)
        pltpu.async_copy(rcv_hbm.at[pl.ds(base, CHUNK)], idxb.at[ib, 1], si)
        pltpu.async_copy(tp_hbm.at[pl.ds(base, CHUNK)], tprows.at[xb],
                         sem_tp[xb])

    def wait_idx(ib):
        pltpu.make_async_copy(snd_hbm.at[pl.ds(base0, CHUNK)],
                              idxb.at[ib, 0], si).wait()
        pltpu.make_async_copy(rcv_hbm.at[pl.ds(base0, CHUNK)],
                              idxb.at[ib, 1], si).wait()

    def wait_tp(xb):
        pltpu.make_async_copy(tp_hbm.at[pl.ds(base0, CHUNK)],
                              tprows.at[xb], sem_tp[xb]).wait()

    def start_gather(ib, xb):
        pltpu.async_copy(x_hbm.at[idxb.at[ib, 0]], xrows.at[xb], sem_g[xb])

    def wait_gather(ib, xb):
        pltpu.make_async_copy(x_hbm.at[idxb.at[ib, 0]], xrows.at[xb],
                              sem_g[xb]).wait()

    def start_scatter(ib, xb):
        pltpu.async_copy(xrows.at[xb], acc.at[idxb.at[ib, 1]], sem_s[xb],
                         add=True)

    def wait_scatter(ib, xb):
        pltpu.make_async_copy(xrows.at[xb], acc.at[idxb.at[ib, 1]],
                              sem_s[xb]).wait()

    def multiply(xb):
        def mul_body(i, carry):
            for j in range(D // 16):
                sl = pl.ds(j * 16, 16)
                xrows[xb, i, sl] = xrows[xb, i, sl] * tprows[xb, i, sl]
            return carry
        lax.fori_loop(0, CHUNK, mul_body, 0)

    def chunk_step(t, ib, xb, prefetch=True, first_guard=None):
        in1 = (ib + 1) % NIDX
        xn = xb ^ 1
        if prefetch:
            start_idx_tp(t + 1, in1, xn)
        wait_tp(xb)
        wait_gather(ib, xb)
        multiply(xb)
        if first_guard is None:
            wait_scatter((ib - 1) % NIDX, xn)
        else:
            @pl.when(first_guard)
            def _():
                wait_scatter((ib - 1) % NIDX, xn)
        if prefetch:
            wait_idx(in1)
            start_gather(in1, xn)
        start_scatter(ib, xb)

    start_idx_tp(0, 0, 0)
    pltpu.sync_copy(zero_hbm, acc.at[pl.ds(s * ROWS_PER_TILE, ROWS_PER_TILE)])
    plsc.subcore_barrier()
    wait_idx(0)
    start_gather(0, 0)

    def quad_body(q, carry):
        chunk_step(4 * q + 0, 0, 0, first_guard=q >= 1)
        chunk_step(4 * q + 1, 1, 1)
        chunk_step(4 * q + 2, 2, 0)
        chunk_step(4 * q + 3, 3, 1)
        return carry

    lax.fori_loop(0, NQ, quad_body, 0)

    wait_tp(0)
    wait_gather(0, 0)
    multiply(0)
    wait_scatter(3, 1)
    start_scatter(0, 0)
    wait_scatter(0, 0)
    plsc.subcore_barrier()

    pltpu.sync_copy(acc.at[pl.ds(s * ROWS_PER_TILE, ROWS_PER_TILE)],
                    out_hbm.at[c, pl.ds(s * ROWS_PER_TILE, ROWS_PER_TILE)])


def _sc_scatter(x, tp_scaled, sender, receiver, zeros):
    mesh = plsc.VectorSubcoreMesh(core_axis_name="c", subcore_axis_name="s")
    f = functools.partial(
        pl.kernel,
        out_type=jax.ShapeDtypeStruct((NUM_SC, N_PAD, D), jnp.float32),
        mesh=mesh,
        scratch_types=[
            pltpu.VMEM((NIDX, 2, CHUNK), jnp.int32),
            pltpu.VMEM((2, CHUNK, D), jnp.float32),
            pltpu.VMEM((2, CHUNK, D), jnp.float32),
            pltpu.VMEM_SHARED((N_PAD, D), jnp.float32),
            pltpu.SemaphoreType.DMA,
            pltpu.SemaphoreType.DMA,
            pltpu.SemaphoreType.DMA,
            pltpu.SemaphoreType.DMA,
            pltpu.SemaphoreType.DMA,
            pltpu.SemaphoreType.DMA,
            pltpu.SemaphoreType.DMA,
        ],
    )(_sc_body)
    return f(x, tp_scaled, sender, receiver, zeros)


def _final_body(parts_ref, na_ref, wlin_ref, wskip_ref, out_ref):
    m = parts_ref[0] + parts_ref[1]
    z = jnp.dot(m, wlin_ref[...], preferred_element_type=jnp.float32)
    z = z * (1.0 / (math.sqrt(D) * AVG_NUM_NEIGHBORS))
    acc = jnp.zeros(out_ref.shape, jnp.float32)
    for v in range(A):
        acc = acc + jnp.dot(
            z, wskip_ref[:, v, :], preferred_element_type=jnp.float32
        ) * na_ref[:, v:v + 1]
    out_ref[...] = acc * (1.0 / math.sqrt(D * A))


def _final(parts, node_attrs, W_lin, W_skip):
    BN = 2000
    grid = N // BN
    return pl.pallas_call(
        _final_body,
        grid=(grid,),
        in_specs=[
            pl.BlockSpec((NUM_SC, BN, D), lambda i: (0, i, 0)),
            pl.BlockSpec((BN, A), lambda i: (i, 0)),
            pl.BlockSpec((D, D), lambda i: (0, 0)),
            pl.BlockSpec((D, A, D), lambda i: (0, 0, 0)),
        ],
        out_specs=pl.BlockSpec((BN, D), lambda i: (i, 0)),
        out_shape=jax.ShapeDtypeStruct((N, D), jnp.float32),
    )(parts, node_attrs, W_lin, W_skip)


def kernel(node_attrs, node_feats, edge_attrs, edge_feats, edge_index,
           W_up, W1, W2, W3, W4, W_lin, W_skip):
    edge_index = edge_index.astype(jnp.int32)
    tp_scaled = _edge_mlp(edge_feats, edge_attrs, W1, W2, W3, W4)
    x = _linear_up(node_feats, W_up)
    zeros = jnp.zeros((ROWS_PER_TILE, D), jnp.float32)
    parts = _sc_scatter(x, tp_scaled, edge_index[0], edge_index[1], zeros)
    return _final(parts, node_attrs, W_lin, W_skip)
